# baseline (speedup 1.0000x reference)
import functools

import jax
import jax.numpy as jnp
from jax import lax
from jax.experimental import pallas as pl
from jax.experimental.pallas import tpu as pltpu
from jax.experimental.pallas import tpu_sc as plsc

N_USERS = 1000000
N_ITEMS = 1000000
EMBED_DIM = 32
BATCH = 16384

NC = 2
NS = 16
L = 16
NW = NC * NS
B_PER_W = BATCH // NW
IDX_CHUNK = 128
N_CHUNKS = B_PER_W // IDX_CHUNK
N_VECS = B_PER_W // L


def _mf_kernel(user_idx_hbm, item_idx_hbm, user_table, item_table,
               user_bias, item_bias, gb_hbm, out_hbm,
               idx_u, idx_i, rows_u, rows_i, bias_u, bias_i, gb_v,
               tbuf, out_v, sem):
    wid = lax.axis_index("s") * NC + lax.axis_index("c")
    base_blk = wid * N_CHUNKS

    pltpu.sync_copy(user_idx_hbm.at[pl.ds(base_blk, N_CHUNKS)], idx_u)
    pltpu.sync_copy(item_idx_hbm.at[pl.ds(base_blk, N_CHUNKS)], idx_i)
    pltpu.sync_copy(gb_hbm, gb_v)

    copies = []
    for j in range(N_CHUNKS):
        s = pl.ds(j * IDX_CHUNK, IDX_CHUNK)
        copies.append(pltpu.async_copy(
            user_table.at[idx_u.at[j]], rows_u.at[s], sem))
        copies.append(pltpu.async_copy(
            item_table.at[idx_i.at[j]], rows_i.at[s], sem))
        copies.append(pltpu.async_copy(
            user_bias.at[idx_u.at[j]], bias_u.at[s], sem))
        copies.append(pltpu.async_copy(
            item_bias.at[idx_i.at[j]], bias_i.at[s], sem))
    for c in copies:
        c.wait()

    gb = gb_v[...]
    lane = lax.iota(jnp.int32, L)
    scatter_idx = [lane * L + b for b in range(L)]
    half = EMBED_DIM // 2

    def body(c, _):
        for b in range(L):
            r = c * L + b
            u0 = rows_u[r, pl.ds(0, half)]
            u1 = rows_u[r, pl.ds(half, half)]
            i0 = rows_i[r, pl.ds(0, half)]
            i1 = rows_i[r, pl.ds(half, half)]
            plsc.store_scatter(tbuf, [scatter_idx[b]], u0 * i0 + u1 * i1)
        acc = tbuf[pl.ds(0, L)]
        for l in range(1, L):
            acc = acc + tbuf[pl.ds(l * L, L)]
        ub = bias_u[pl.ds(c * L, L)]
        ib = bias_i[pl.ds(c * L, L)]
        p = acc + ub + ib + gb
        out_v[pl.ds(c * L, L)] = 1.0 / (1.0 + jnp.exp(-p))
        return _

    lax.fori_loop(0, N_VECS, body, None)

    pltpu.sync_copy(out_v, out_hbm.at[pl.ds(wid * B_PER_W, B_PER_W)])


@functools.partial(jax.jit, static_argnums=())
def kernel(user_idx, item_idx, user_table, item_table, user_bias_table,
           item_bias_table, global_bias):
    mesh = plsc.VectorSubcoreMesh(core_axis_name="c", subcore_axis_name="s")
    run = pl.kernel(
        _mf_kernel,
        mesh=mesh,
        compiler_params=pltpu.CompilerParams(
            needs_layout_passes=False, use_tc_tiling_on_sc=False),
        out_type=jax.ShapeDtypeStruct((BATCH,), jnp.float32),
        scratch_types=[
            pltpu.VMEM((N_CHUNKS, IDX_CHUNK), jnp.int32),
            pltpu.VMEM((N_CHUNKS, IDX_CHUNK), jnp.int32),
            pltpu.VMEM((B_PER_W, EMBED_DIM), jnp.float32),
            pltpu.VMEM((B_PER_W, EMBED_DIM), jnp.float32),
            pltpu.VMEM((B_PER_W,), jnp.float32),
            pltpu.VMEM((B_PER_W,), jnp.float32),
            pltpu.VMEM((L,), jnp.float32),
            pltpu.VMEM((L * L,), jnp.float32),
            pltpu.VMEM((B_PER_W,), jnp.float32),
            pltpu.SemaphoreType.DMA,
        ],
    )
    uidx = user_idx.astype(jnp.int32).reshape(BATCH // IDX_CHUNK, IDX_CHUNK)
    iidx = item_idx.astype(jnp.int32).reshape(BATCH // IDX_CHUNK, IDX_CHUNK)
    gb16 = jnp.broadcast_to(global_bias.astype(jnp.float32), (L,))
    return run(uidx, iidx, user_table, item_table,
               user_bias_table.reshape(N_USERS), item_bias_table.reshape(N_ITEMS),
               gb16)
